# baseline (speedup 1.0000x reference)
import functools

import jax
import jax.numpy as jnp
from jax import lax
from jax.experimental import pallas as pl
from jax.experimental.pallas import tpu as pltpu
from jax.experimental.pallas import tpu_sc as plsc

U = 100000
B = 16384
D = 64
H1 = 64
H2 = 32
EPS = 1e-5

NC = 2
NS = 16
NW = NC * NS
B_PER_W = B // NW
CHUNK = 256
NCHUNK = B_PER_W // CHUNK

RB = 1000
BK = 1024


def _repack_body(umf_r, umlp_r, imf_r, imlp_r, outu_r, outi_r):
    outu_r[:, 0:D] = umf_r[:]
    outu_r[:, D:2 * D] = umlp_r[:]
    outi_r[:, 0:D] = imf_r[:]
    outi_r[:, D:2 * D] = imlp_r[:]


def _tc_repack(user_mf, user_mlp, item_mf, item_mlp):
    bs_in = pl.BlockSpec((RB, D), lambda i: (i, 0))
    bs_out = pl.BlockSpec((RB, 2 * D), lambda i: (i, 0))
    return pl.pallas_call(
        _repack_body,
        grid=(U // RB,),
        in_specs=[bs_in, bs_in, bs_in, bs_in],
        out_specs=[bs_out, bs_out],
        out_shape=[
            jax.ShapeDtypeStruct((U, 2 * D), jnp.float32),
            jax.ShapeDtypeStruct((U, 2 * D), jnp.float32),
        ],
    )(user_mf, user_mlp, item_mf, item_mlp)


def _sc_gather(users, items, cat_u, cat_i):
    mesh = plsc.VectorSubcoreMesh(core_axis_name="c", subcore_axis_name="s")

    @functools.partial(
        pl.kernel,
        mesh=mesh,
        out_type=(
            jax.ShapeDtypeStruct((B, 2 * D), jnp.float32),
            jax.ShapeDtypeStruct((B, 2 * D), jnp.float32),
        ),
        scratch_types=[
            pltpu.VMEM((CHUNK,), jnp.int32),
            pltpu.VMEM((CHUNK,), jnp.int32),
            pltpu.VMEM((CHUNK, 2 * D), jnp.float32),
            pltpu.VMEM((CHUNK, 2 * D), jnp.float32),
            pltpu.SemaphoreType.DMA,
        ],
    )
    def sc_kernel(users_h, items_h, catu_h, cati_h,
                  gu_o, gi_o, idxu, idxi, bufu, bufi, sem):
        wid = lax.axis_index("s") * NC + lax.axis_index("c")
        for c in range(NCHUNK):
            gbase = wid * B_PER_W + c * CHUNK
            pltpu.sync_copy(users_h.at[pl.ds(gbase, CHUNK)], idxu)
            pltpu.sync_copy(items_h.at[pl.ds(gbase, CHUNK)], idxi)
            cps = [
                pltpu.async_copy(catu_h.at[idxu], bufu, sem),
                pltpu.async_copy(cati_h.at[idxi], bufi, sem),
            ]
            for cp in cps:
                cp.wait()
            pltpu.sync_copy(bufu, gu_o.at[pl.ds(gbase, CHUNK)])
            pltpu.sync_copy(bufi, gi_o.at[pl.ds(gbase, CHUNK)])

    return sc_kernel(users, items, cat_u, cat_i)


def _tc_body(gu_r, gi_r, w1a_r, w1b_r, b1_r, w2_r, b2_r,
             wmf_r, wmlp_r, c0_r, out_r):
    ug = gu_r[:, D:2 * D]
    ig = gi_r[:, D:2 * D]
    h1 = jnp.dot(ug, w1a_r[:], preferred_element_type=jnp.float32)
    h1 = h1 + jnp.dot(ig, w1b_r[:], preferred_element_type=jnp.float32)
    h1 = jnp.maximum(h1 + b1_r[:], 0.0)
    h2 = jnp.dot(h1, w2_r[:], preferred_element_type=jnp.float32) + b2_r[:]
    h2 = jnp.maximum(h2, 0.0)
    prod = gu_r[:, 0:D] * gi_r[:, 0:D]
    mf = jnp.sum(prod * wmf_r[:], axis=1, keepdims=True)
    ml = jnp.sum(h2 * wmlp_r[:], axis=1, keepdims=True)
    out_r[:] = mf + ml + c0_r[0, 0]


def _tc_head(gu, gi, w1a, w1b, b1, w2f, b2f, wmf, wmlp, c0):
    bs_rows = pl.BlockSpec((BK, 2 * D), lambda i: (i, 0))

    def bs_full(shape):
        return pl.BlockSpec(shape, lambda i: (0,) * len(shape))

    return pl.pallas_call(
        _tc_body,
        grid=(B // BK,),
        in_specs=[
            bs_rows, bs_rows,
            bs_full((D, H1)), bs_full((D, H1)), bs_full((1, H1)),
            bs_full((H1, H2)), bs_full((1, H2)),
            bs_full((1, D)), bs_full((1, H2)), bs_full((1, 1)),
        ],
        out_specs=pl.BlockSpec((BK, 1), lambda i: (i, 0)),
        out_shape=jax.ShapeDtypeStruct((B, 1), jnp.float32),
    )(gu, gi, w1a, w1b, b1, w2f, b2f, wmf, wmlp, c0)


def kernel(users, items, user_mf, item_mf, user_mlp, item_mlp,
           W1, b1, g1, be1, m1, v1, W2, b2, g2, be2, m2, v2, Wp, bp):
    users = users.astype(jnp.int32)
    items = items.astype(jnp.int32)

    cat_u, cat_i = _tc_repack(user_mf, user_mlp, item_mf, item_mlp)
    gu, gi = _sc_gather(users, items, cat_u, cat_i)

    s1 = g1 / jnp.sqrt(v1 + EPS)
    t1 = be1 - m1 * s1
    s2 = g2 / jnp.sqrt(v2 + EPS)
    t2 = be2 - m2 * s2
    w1a = W1[:D]
    w1b = W1[D:]
    w2f = s1[:, None] * W2
    b2f = b2 + t1 @ W2
    wmf = Wp[:D, 0]
    wmlp = s2 * Wp[D:, 0]
    c0 = t2 @ Wp[D:, 0] + bp[0]

    out = _tc_head(gu, gi, w1a, w1b,
                   b1.reshape(1, H1), w2f, b2f.reshape(1, H2),
                   wmf.reshape(1, D), wmlp.reshape(1, H2),
                   c0.reshape(1, 1))
    return out[:, 0]


# speedup vs baseline: 2.1943x; 2.1943x over previous
import functools

import jax
import jax.numpy as jnp
from jax import lax
from jax.experimental import pallas as pl
from jax.experimental.pallas import tpu as pltpu
from jax.experimental.pallas import tpu_sc as plsc

U = 100000
B = 16384
D = 64
H1 = 64
H2 = 32
EPS = 1e-5

NC = 2
NS = 16
NW = NC * NS
FPW = 4 * D // NW
HALF = B // 2

BKC = 2048


def _sc_gather(users, items, umf_t, imf_t, umlp_t, imlp_t):
    mesh = plsc.VectorSubcoreMesh(core_axis_name="c", subcore_axis_name="s")

    @functools.partial(
        pl.kernel,
        mesh=mesh,
        compiler_params=pltpu.CompilerParams(needs_layout_passes=False),
        out_type=jax.ShapeDtypeStruct((4 * D, B), jnp.float32),
        scratch_types=[
            pltpu.VMEM((U,), jnp.float32),
            pltpu.VMEM((HALF,), jnp.int32),
            pltpu.VMEM((HALF,), jnp.float32),
        ],
    )
    def sc_kernel(users_h, items_h, umf_h, imf_h, umlp_h, imlp_h,
                  out_o, colbuf, idx_v, outcol):
        wid = lax.axis_index("s") * NC + lax.axis_index("c")
        f0 = wid * 2
        tables = [(umf_h, users_h), (imf_h, items_h),
                  (umlp_h, users_h), (imlp_h, items_h)]
        for t, (tbl, idx_h) in enumerate(tables):
            for f in range(2):
                col = f0 + f
                pltpu.sync_copy(tbl.at[col], colbuf)
                for half in range(2):
                    pltpu.sync_copy(idx_h.at[pl.ds(half * HALF, HALF)], idx_v)

                    def gather_body(v, carry):
                        iv = idx_v[pl.ds(v * 16, 16)]
                        outcol[pl.ds(v * 16, 16)] = plsc.load_gather(
                            colbuf, [iv])
                        return carry

                    lax.fori_loop(0, HALF // 16, gather_body, 0)
                    pltpu.sync_copy(
                        outcol,
                        out_o.at[t * D + col, pl.ds(half * HALF, HALF)])

    return sc_kernel(users, items, umf_t, imf_t, umlp_t, imlp_t)


def _tc_body(g_r, w1at_r, w1bt_r, b1_r, w2ft_r, b2f_r,
             wmf_r, wmlp_r, c0_r, out_r):
    g = g_r[:]
    umf_g = g[0:D]
    imf_g = g[D:2 * D]
    ug_g = g[2 * D:3 * D]
    ig_g = g[3 * D:4 * D]
    h1 = jnp.dot(w1at_r[:], ug_g, preferred_element_type=jnp.float32)
    h1 = h1 + jnp.dot(w1bt_r[:], ig_g, preferred_element_type=jnp.float32)
    h1 = jnp.maximum(h1 + b1_r[:], 0.0)
    h2 = jnp.dot(w2ft_r[:], h1, preferred_element_type=jnp.float32) + b2f_r[:]
    h2 = jnp.maximum(h2, 0.0)
    prod = umf_g * imf_g
    mf = jnp.dot(wmf_r[:], prod, preferred_element_type=jnp.float32)
    ml = jnp.dot(wmlp_r[:], h2, preferred_element_type=jnp.float32)
    out_r[:] = mf + ml + c0_r[0, 0]


def _tc_head(g, w1at, w1bt, b1c, w2ft, b2fc, wmf_row, wmlp_row, c0):
    def bs_full(shape):
        return pl.BlockSpec(shape, lambda i: (0,) * len(shape))

    return pl.pallas_call(
        _tc_body,
        grid=(B // BKC,),
        in_specs=[
            pl.BlockSpec((4 * D, BKC), lambda i: (0, i)),
            bs_full((D, H1)), bs_full((D, H1)), bs_full((H1, 1)),
            bs_full((H2, H1)), bs_full((H2, 1)),
            bs_full((1, D)), bs_full((1, H2)), bs_full((1, 1)),
        ],
        out_specs=pl.BlockSpec((1, BKC), lambda i: (0, i)),
        out_shape=jax.ShapeDtypeStruct((1, B), jnp.float32),
    )(g, w1at, w1bt, b1c, w2ft, b2fc, wmf_row, wmlp_row, c0)


def kernel(users, items, user_mf, item_mf, user_mlp, item_mlp,
           W1, b1, g1, be1, m1, v1, W2, b2, g2, be2, m2, v2, Wp, bp):
    users = users.astype(jnp.int32)
    items = items.astype(jnp.int32)

    g = _sc_gather(users, items,
                   user_mf.T, item_mf.T, user_mlp.T, item_mlp.T)

    s1 = g1 / jnp.sqrt(v1 + EPS)
    t1 = be1 - m1 * s1
    s2 = g2 / jnp.sqrt(v2 + EPS)
    t2 = be2 - m2 * s2
    w1at = W1[:D].T
    w1bt = W1[D:].T
    w2ft = (s1[:, None] * W2).T
    b2f = b2 + t1 @ W2
    wmf = Wp[:D, 0]
    wmlp = s2 * Wp[D:, 0]
    c0 = t2 @ Wp[D:, 0] + bp[0]

    out = _tc_head(g, w1at, w1bt, b1.reshape(H1, 1),
                   w2ft, b2f.reshape(H2, 1),
                   wmf.reshape(1, D), wmlp.reshape(1, H2),
                   c0.reshape(1, 1))
    return out[0]
